# baseline (speedup 1.0000x reference)
from math import sqrt

import functools

import jax
import jax.numpy as jnp
from jax import lax
from jax.experimental import pallas as pl
from jax.experimental.pallas import tpu as pltpu

_MASK_VALUE = -1e30


def _fused_attn_kernel(x_ref, wqkv_ref, bqkv_ref, wo_ref, bo_ref,
                       y_ref, a_ref, qkv_scr, m_sc, l_sc, acc_sc,
                       *, n_heads, d_keys, lt, nc, scale):
    i = pl.program_id(1)
    H, dk = n_heads, d_keys
    hd = H * dk
    L = x_ref.shape[1]
    ct = L // nc

    @pl.when(i == 0)
    def _project():
        qkv_scr[...] = (
            jnp.dot(x_ref[0], wqkv_ref[...],
                    preferred_element_type=jnp.float32)
            + bqkv_ref[...]
        )

    q_all = qkv_scr[pl.ds(i * lt, lt), 0:hd] * scale
    m_sc[...] = jnp.full_like(m_sc, -jnp.inf)
    l_sc[...] = jnp.zeros_like(l_sc)
    acc_sc[...] = jnp.zeros_like(acc_sc)

    diag_mask = (lax.broadcasted_iota(jnp.int32, (lt, ct), 1) >
                 lax.broadcasted_iota(jnp.int32, (lt, ct), 0))

    def _qk_chunk(h, j, masked):
        q = q_all[:, h * dk:(h + 1) * dk]
        k = qkv_scr[j * ct:(j + 1) * ct,
                    hd + h * dk: hd + (h + 1) * dk]
        s = lax.dot_general(q, k, (((1,), (1,)), ((), ())),
                            preferred_element_type=jnp.float32)
        if masked:
            s = jnp.where(diag_mask, _MASK_VALUE, s)
        m_prev = m_sc[h]
        m_new = jnp.maximum(m_prev, jnp.max(s, axis=-1, keepdims=True))
        l_sc[h] = (l_sc[h] * jnp.exp(m_prev - m_new)
                   + jnp.sum(jnp.exp(s - m_new), axis=-1, keepdims=True))
        m_sc[h] = m_new
        a_ref[0, h, :, j * ct:(j + 1) * ct] = s

    for j in range(nc):
        @pl.when(j < i)
        def _(j=j):
            for h in range(H):
                _qk_chunk(h, j, False)

        @pl.when(j == i)
        def _(j=j):
            for h in range(H):
                _qk_chunk(h, j, True)

        @pl.when(j > i)
        def _(j=j):
            z = jnp.zeros((lt, ct), a_ref.dtype)
            for h in range(H):
                a_ref[0, h, :, j * ct:(j + 1) * ct] = z

    m_sc[...] = m_sc[...] + jnp.log(l_sc[...])

    for j in range(nc):
        @pl.when(j <= i)
        def _(j=j):
            for h in range(H):
                s = a_ref[0, h, :, j * ct:(j + 1) * ct]
                p = jnp.exp(s - m_sc[h])
                a_ref[0, h, :, j * ct:(j + 1) * ct] = p
                v = qkv_scr[j * ct:(j + 1) * ct,
                            2 * hd + h * dk: 2 * hd + (h + 1) * dk]
                acc_sc[:, h * dk:(h + 1) * dk] += lax.dot_general(
                    p, v, (((1,), (0,)), ((), ())),
                    preferred_element_type=jnp.float32)

    y_ref[0] = (jnp.dot(acc_sc[...], wo_ref[...],
                        preferred_element_type=jnp.float32)
                + bo_ref[...]).astype(y_ref.dtype)


def kernel(x, wqkv3, bqkv3, wo3, bo):
    B, L, d_model = x.shape
    G, _, dk = wqkv3.shape
    H = G // 3
    hd = H * dk
    lt = 128 if L % 128 == 0 else L
    nc = L // lt
    scale = 1.0 / sqrt(dk)

    wqkv_flat = jnp.transpose(wqkv3, (1, 0, 2)).reshape(d_model, G * dk)
    bqkv_flat = bqkv3.reshape(1, G * dk)
    wo_flat = wo3.reshape(hd, d_model)

    kern = functools.partial(_fused_attn_kernel, n_heads=H, d_keys=dk,
                             lt=lt, nc=nc, scale=scale)
    y, attn = pl.pallas_call(
        kern,
        out_shape=(
            jax.ShapeDtypeStruct((B, L, d_model), x.dtype),
            jax.ShapeDtypeStruct((B, H, L, L), x.dtype),
        ),
        grid_spec=pltpu.PrefetchScalarGridSpec(
            num_scalar_prefetch=0,
            grid=(B, L // lt),
            in_specs=[
                pl.BlockSpec((1, L, d_model), lambda b, i: (b, 0, 0)),
                pl.BlockSpec((d_model, G * dk), lambda b, i: (0, 0)),
                pl.BlockSpec((1, G * dk), lambda b, i: (0, 0)),
                pl.BlockSpec((hd, d_model), lambda b, i: (0, 0)),
                pl.BlockSpec((1, d_model), lambda b, i: (0, 0)),
            ],
            out_specs=(
                pl.BlockSpec((1, lt, d_model), lambda b, i: (b, i, 0)),
                pl.BlockSpec((1, H, lt, L), lambda b, i: (b, 0, i, 0)),
            ),
            scratch_shapes=[
                pltpu.VMEM((L, G * dk), jnp.float32),
                pltpu.VMEM((H, lt, 1), jnp.float32),
                pltpu.VMEM((H, lt, 1), jnp.float32),
                pltpu.VMEM((lt, hd), jnp.float32),
            ],
        ),
        compiler_params=pltpu.CompilerParams(
            dimension_semantics=("parallel", "arbitrary"),
            vmem_limit_bytes=60 * 1024 * 1024,
        ),
    )(x, wqkv_flat, bqkv_flat, wo_flat, bo)
    return y, attn


# speedup vs baseline: 2.5320x; 2.5320x over previous
from math import log2, e as _e, sqrt

import functools

import jax
import jax.numpy as jnp
from jax import lax
from jax.experimental import pallas as pl
from jax.experimental.pallas import tpu as pltpu

_MASK_VALUE = -1e30


def _fused_attn_kernel(x_ref, wqkv_ref, bqkv_ref, wo_ref, bo_ref,
                       y_ref, a_ref, qkv_scr, acc_sc,
                       *, n_heads, d_keys, lt, d_model):
    i = pl.program_id(1)
    H, dk = n_heads, d_keys
    hd = H * dk
    L = x_ref.shape[1]

    @pl.when(i == 0)
    def _project():
        qkv_scr[...] = (
            jnp.dot(x_ref[0], wqkv_ref[...],
                    preferred_element_type=jnp.float32)
            + bqkv_ref[...]
        )

    def _attend(width):
        q_all = qkv_scr[pl.ds(i * lt, lt), 0:hd]
        row = i * lt + lax.broadcasted_iota(jnp.int32, (lt, width), 0)
        col = lax.broadcasted_iota(jnp.int32, (lt, width), 1)
        causal = col > row
        for h in range(H):
            q = q_all[:, h * dk:(h + 1) * dk]
            k = qkv_scr[0:width, hd + h * dk: hd + (h + 1) * dk]
            v = qkv_scr[0:width, 2 * hd + h * dk: 2 * hd + (h + 1) * dk]
            s = lax.dot_general(q, k, (((1,), (1,)), ((), ())),
                                preferred_element_type=jnp.float32)
            s = jnp.where(causal, _MASK_VALUE, s)
            m = jnp.max(s, axis=-1, keepdims=True)
            p = jnp.exp2(s - m)
            denom = jnp.sum(p, axis=-1, keepdims=True)
            a = p * (1.0 / denom)
            a_ref[0, h, :, 0:width] = a.astype(a_ref.dtype)
            if width < L:
                a_ref[0, h, :, width:L] = jnp.zeros((lt, L - width),
                                                    a_ref.dtype)
            acc_sc[:, h * dk:(h + 1) * dk] = lax.dot_general(
                a, v, (((1,), (0,)), ((), ())),
                preferred_element_type=jnp.float32)
        y_ref[0] = (jnp.dot(acc_sc[...], wo_ref[...],
                            preferred_element_type=jnp.float32)
                    + bo_ref[...]).astype(y_ref.dtype)

    half = (L // lt) // 2
    if half >= 1:
        @pl.when(i < half)
        def _narrow():
            _attend(half * lt)

        @pl.when(i >= half)
        def _wide():
            _attend(L)
    else:
        _attend(L)


def kernel(x, wqkv3, bqkv3, wo3, bo):
    B, L, d_model = x.shape
    G, _, dk = wqkv3.shape
    H = G // 3
    hd = H * dk
    lt = 128 if L % 128 == 0 else L
    scale = log2(_e) / sqrt(dk)

    wqkv_flat = jnp.transpose(wqkv3, (1, 0, 2)).reshape(d_model, G * dk)
    bqkv_flat = bqkv3.reshape(1, G * dk)
    qscale = jnp.concatenate(
        [jnp.full((1, hd), scale, wqkv_flat.dtype),
         jnp.ones((1, 2 * hd), wqkv_flat.dtype)], axis=1)
    wqkv_flat = wqkv_flat * qscale
    bqkv_flat = bqkv_flat * qscale
    wo_flat = wo3.reshape(hd, d_model)

    kern = functools.partial(_fused_attn_kernel, n_heads=H, d_keys=dk,
                             lt=lt, d_model=d_model)
    y, attn = pl.pallas_call(
        kern,
        out_shape=(
            jax.ShapeDtypeStruct((B, L, d_model), x.dtype),
            jax.ShapeDtypeStruct((B, H, L, L), x.dtype),
        ),
        grid_spec=pltpu.PrefetchScalarGridSpec(
            num_scalar_prefetch=0,
            grid=(B, L // lt),
            in_specs=[
                pl.BlockSpec((1, L, d_model), lambda b, i: (b, 0, 0)),
                pl.BlockSpec((d_model, G * dk), lambda b, i: (0, 0)),
                pl.BlockSpec((1, G * dk), lambda b, i: (0, 0)),
                pl.BlockSpec((hd, d_model), lambda b, i: (0, 0)),
                pl.BlockSpec((1, d_model), lambda b, i: (0, 0)),
            ],
            out_specs=(
                pl.BlockSpec((1, lt, d_model), lambda b, i: (b, i, 0)),
                pl.BlockSpec((1, H, lt, L), lambda b, i: (b, 0, i, 0)),
            ),
            scratch_shapes=[
                pltpu.VMEM((L, G * dk), jnp.float32),
                pltpu.VMEM((lt, hd), jnp.float32),
            ],
        ),
        compiler_params=pltpu.CompilerParams(
            dimension_semantics=("parallel", "arbitrary"),
            vmem_limit_bytes=60 * 1024 * 1024,
        ),
    )(x, wqkv_flat, bqkv_flat, wo_flat, bo)
    return y, attn
